# baseline (speedup 1.0000x reference)
import functools

import jax
import jax.numpy as jnp
from jax import lax
from jax.experimental import pallas as pl
from jax.experimental.pallas import tpu as pltpu
from jax.experimental.pallas import tpu_sc as plsc

N_NODES = 10000
D_FEAT = 128
N_EDGES = 320000

N_CORES = 2
N_TILES = 16
N_WORKERS = N_CORES * N_TILES

N_PAD = 10240
CHUNK = 125
EDGES_PER_TILE = N_EDGES // N_WORKERS
N_CHUNKS = EDGES_PER_TILE // CHUNK
SEG = 16
N_SEGS = N_CHUNKS // SEG
SEG_PAIRS = SEG // 2
ROWS_PER_TILE = N_PAD // N_TILES
ZROWS = 80


def _sc_partials(x, idx5):
    mesh = plsc.VectorSubcoreMesh(core_axis_name="c", subcore_axis_name="s")

    @functools.partial(
        pl.kernel,
        mesh=mesh,
        out_type=jax.ShapeDtypeStruct((N_CORES, N_PAD, D_FEAT), jnp.float32),
        scratch_types=[
            pltpu.VMEM((SEG, CHUNK), jnp.int32),
            pltpu.VMEM((SEG, CHUNK), jnp.int32),
            pltpu.VMEM((CHUNK, D_FEAT), jnp.float32),
            pltpu.VMEM((CHUNK, D_FEAT), jnp.float32),
            pltpu.VMEM_SHARED((N_PAD, D_FEAT), jnp.float32),
            pltpu.SemaphoreType.DMA,
            pltpu.SemaphoreType.DMA,
            pltpu.SemaphoreType.DMA,
            pltpu.SemaphoreType.DMA,
        ],
    )
    def k(
        x_hbm, idx_hbm, out_hbm,
        src_t, dst_t, r0, r1, acc,
        gs0, gs1, ss0, ss1,
    ):
        c = lax.axis_index("c")
        s = lax.axis_index("s")
        wid = s * N_CORES + c
        rows = (r0, r1)
        gs = (gs0, gs1)
        ss = (ss0, ss1)

        zv = jnp.zeros((16,), jnp.float32)

        def zbody(i, carry):
            for kk in range(8):
                r0[i, pl.ds(kk * 16, 16)] = zv
            return carry

        lax.fori_loop(0, ZROWS, zbody, 0)
        for kk in range(ROWS_PER_TILE // ZROWS):
            pltpu.sync_copy(
                r0.at[pl.ds(0, ZROWS)],
                acc.at[pl.ds(s * ROWS_PER_TILE + kk * ZROWS, ZROWS)],
            )
        plsc.subcore_barrier()

        def fire_gather(j, b):
            pltpu.async_copy(x_hbm.at[src_t.at[j]], rows[b], gs[b])

        def wait_gather(j, b):
            pltpu.make_async_copy(x_hbm.at[src_t.at[j]], rows[b], gs[b]).wait()

        def fire_scatter(j, b):
            pltpu.async_copy(rows[b], acc.at[dst_t.at[j]], ss[b], add=True)

        def wait_scatter(j, b):
            pltpu.make_async_copy(rows[b], acc.at[dst_t.at[j]], ss[b]).wait()

        for seg in range(N_SEGS):
            pltpu.sync_copy(idx_hbm.at[0, wid, seg], src_t)
            pltpu.sync_copy(idx_hbm.at[1, wid, seg], dst_t)

            fire_gather(0, 0)

            def body(i, carry):
                a = 2 * i
                b = a + 1
                wait_gather(a, 0)
                @pl.when(i > 0)
                def _():
                    wait_scatter(a - 1, 1)
                fire_gather(b, 1)
                fire_scatter(a, 0)
                wait_gather(b, 1)
                wait_scatter(a, 0)
                @pl.when(i < SEG_PAIRS - 1)
                def _():
                    fire_gather(a + 2, 0)
                fire_scatter(b, 1)
                return carry

            lax.fori_loop(0, SEG_PAIRS, body, 0)

            wait_scatter(SEG - 1, 1)

        plsc.subcore_barrier()
        pltpu.sync_copy(
            acc.at[pl.ds(s * ROWS_PER_TILE, ROWS_PER_TILE)],
            out_hbm.at[c, pl.ds(s * ROWS_PER_TILE, ROWS_PER_TILE)],
        )

    return k(x, idx5)


COMB_ROWS = N_PAD // N_WORKERS
COMB_TAIL = N_NODES - (N_WORKERS - 1) * COMB_ROWS


def _combine(partials):
    mesh = plsc.VectorSubcoreMesh(core_axis_name="c", subcore_axis_name="s")

    @functools.partial(
        pl.kernel,
        mesh=mesh,
        out_type=jax.ShapeDtypeStruct((N_NODES, D_FEAT), jnp.float32),
        scratch_types=[
            pltpu.VMEM((COMB_ROWS, D_FEAT), jnp.float32),
            pltpu.VMEM((COMB_ROWS, D_FEAT), jnp.float32),
            pltpu.SemaphoreType.DMA,
            pltpu.SemaphoreType.DMA,
        ],
    )
    def k(p_hbm, out_hbm, a0, a1, sem0, sem1):
        c = lax.axis_index("c")
        s = lax.axis_index("s")
        wid = s * N_CORES + c
        base = wid * COMB_ROWS
        cp0 = pltpu.async_copy(p_hbm.at[0, pl.ds(base, COMB_ROWS)], a0, sem0)
        cp1 = pltpu.async_copy(p_hbm.at[1, pl.ds(base, COMB_ROWS)], a1, sem1)
        cp0.wait()
        cp1.wait()

        def body(i, carry):
            for kk in range(8):
                sl = pl.ds(kk * 16, 16)
                a0[i, sl] = a0[i, sl] + a1[i, sl]
            return carry

        lax.fori_loop(0, COMB_ROWS, body, 0)

        @pl.when(wid < N_WORKERS - 1)
        def _():
            pltpu.sync_copy(a0, out_hbm.at[pl.ds(base, COMB_ROWS)])

        @pl.when(wid == N_WORKERS - 1)
        def _():
            pltpu.sync_copy(
                a0.at[pl.ds(0, COMB_TAIL)],
                out_hbm.at[pl.ds(base, COMB_TAIL)],
            )

    return k(partials)


def kernel(x, edge_index):
    idx5 = edge_index.reshape(2, N_WORKERS, N_SEGS, SEG, CHUNK)
    partials = _sc_partials(x, idx5)
    return _combine(partials)


# speedup vs baseline: 1.0912x; 1.0912x over previous
import functools

import jax
import jax.numpy as jnp
from jax import lax
from jax.experimental import pallas as pl
from jax.experimental.pallas import tpu as pltpu
from jax.experimental.pallas import tpu_sc as plsc

N_NODES = 10000
D_FEAT = 128
N_EDGES = 320000

N_CORES = 2
N_TILES = 16
N_WORKERS = N_CORES * N_TILES

N_PAD = 10240
CHUNK = 50
EDGES_PER_TILE = N_EDGES // N_WORKERS
N_CHUNKS = EDGES_PER_TILE // CHUNK
SEG = 25
N_SEGS = N_CHUNKS // SEG
NBUF = 4
RING_TRIPS = (SEG - 5) // NBUF
ROWS_PER_TILE = N_PAD // N_TILES
ZROWS = 40


def _sc_partials(x, idx5):
    mesh = plsc.VectorSubcoreMesh(core_axis_name="c", subcore_axis_name="s")

    @functools.partial(
        pl.kernel,
        mesh=mesh,
        out_type=jax.ShapeDtypeStruct((N_CORES, N_PAD, D_FEAT), jnp.float32),
        scratch_types=[
            pltpu.VMEM((SEG, CHUNK), jnp.int32),
            pltpu.VMEM((SEG, CHUNK), jnp.int32),
            pltpu.VMEM((CHUNK, D_FEAT), jnp.float32),
            pltpu.VMEM((CHUNK, D_FEAT), jnp.float32),
            pltpu.VMEM((CHUNK, D_FEAT), jnp.float32),
            pltpu.VMEM((CHUNK, D_FEAT), jnp.float32),
            pltpu.VMEM_SHARED((N_PAD, D_FEAT), jnp.float32),
            pltpu.SemaphoreType.DMA,
            pltpu.SemaphoreType.DMA,
            pltpu.SemaphoreType.DMA,
            pltpu.SemaphoreType.DMA,
            pltpu.SemaphoreType.DMA,
            pltpu.SemaphoreType.DMA,
            pltpu.SemaphoreType.DMA,
            pltpu.SemaphoreType.DMA,
        ],
    )
    def k(
        x_hbm, idx_hbm, out_hbm,
        src_t, dst_t, r0, r1, r2, r3, acc,
        gs0, gs1, gs2, gs3, ss0, ss1, ss2, ss3,
    ):
        c = lax.axis_index("c")
        s = lax.axis_index("s")
        wid = s * N_CORES + c
        rows = (r0, r1, r2, r3)
        gs = (gs0, gs1, gs2, gs3)
        ss = (ss0, ss1, ss2, ss3)

        zv = jnp.zeros((16,), jnp.float32)

        def zbody(i, carry):
            for kk in range(8):
                r0[i, pl.ds(kk * 16, 16)] = zv
            return carry

        lax.fori_loop(0, ZROWS, zbody, 0)
        for kk in range(ROWS_PER_TILE // ZROWS):
            pltpu.sync_copy(
                r0.at[pl.ds(0, ZROWS)],
                acc.at[pl.ds(s * ROWS_PER_TILE + kk * ZROWS, ZROWS)],
            )
        plsc.subcore_barrier()

        def fire_gather(j, b):
            pltpu.async_copy(x_hbm.at[src_t.at[j]], rows[b], gs[b])

        def wait_gather(j, b):
            pltpu.make_async_copy(x_hbm.at[src_t.at[j]], rows[b], gs[b]).wait()

        def fire_scatter(j, b):
            pltpu.async_copy(rows[b], acc.at[dst_t.at[j]], ss[b], add=True)

        def wait_scatter(j, b):
            pltpu.make_async_copy(rows[b], acc.at[dst_t.at[j]], ss[b]).wait()

        for seg in range(N_SEGS):
            pltpu.sync_copy(idx_hbm.at[0, wid, seg], src_t)
            pltpu.sync_copy(idx_hbm.at[1, wid, seg], dst_t)

            fire_gather(0, 0)
            fire_gather(1, 1)
            fire_gather(2, 2)

            def body(t, carry):
                for kk in range(NBUF):
                    j = NBUF * t + kk
                    wait_gather(j, kk)
                    fire_scatter(j, kk)
                    prev = (kk + NBUF - 1) % NBUF
                    if kk == 0:
                        @pl.when(t > 0)
                        def _():
                            wait_scatter(j - 1, prev)
                    else:
                        wait_scatter(j - 1, prev)
                    fire_gather(j + NBUF - 1, prev)
                return carry

            lax.fori_loop(0, RING_TRIPS, body, 0)

            for j in range(NBUF * RING_TRIPS, SEG):
                b = j % NBUF
                prev = (b + NBUF - 1) % NBUF
                wait_gather(j, b)
                fire_scatter(j, b)
                wait_scatter(j - 1, prev)
                if j + NBUF - 1 < SEG:
                    fire_gather(j + NBUF - 1, prev)
            wait_scatter(SEG - 1, (SEG - 1) % NBUF)

        plsc.subcore_barrier()
        pltpu.sync_copy(
            acc.at[pl.ds(s * ROWS_PER_TILE, ROWS_PER_TILE)],
            out_hbm.at[c, pl.ds(s * ROWS_PER_TILE, ROWS_PER_TILE)],
        )

    return k(x, idx5)


COMB_ROWS = N_PAD // N_WORKERS
COMB_TAIL = N_NODES - (N_WORKERS - 1) * COMB_ROWS


def _combine(partials):
    mesh = plsc.VectorSubcoreMesh(core_axis_name="c", subcore_axis_name="s")

    @functools.partial(
        pl.kernel,
        mesh=mesh,
        out_type=jax.ShapeDtypeStruct((N_NODES, D_FEAT), jnp.float32),
        scratch_types=[
            pltpu.VMEM((COMB_ROWS, D_FEAT), jnp.float32),
            pltpu.VMEM((COMB_ROWS, D_FEAT), jnp.float32),
            pltpu.SemaphoreType.DMA,
            pltpu.SemaphoreType.DMA,
        ],
    )
    def k(p_hbm, out_hbm, a0, a1, sem0, sem1):
        c = lax.axis_index("c")
        s = lax.axis_index("s")
        wid = s * N_CORES + c
        base = wid * COMB_ROWS
        cp0 = pltpu.async_copy(p_hbm.at[0, pl.ds(base, COMB_ROWS)], a0, sem0)
        cp1 = pltpu.async_copy(p_hbm.at[1, pl.ds(base, COMB_ROWS)], a1, sem1)
        cp0.wait()
        cp1.wait()

        def body(i, carry):
            for kk in range(8):
                sl = pl.ds(kk * 16, 16)
                a0[i, sl] = a0[i, sl] + a1[i, sl]
            return carry

        lax.fori_loop(0, COMB_ROWS, body, 0)

        @pl.when(wid < N_WORKERS - 1)
        def _():
            pltpu.sync_copy(a0, out_hbm.at[pl.ds(base, COMB_ROWS)])

        @pl.when(wid == N_WORKERS - 1)
        def _():
            pltpu.sync_copy(
                a0.at[pl.ds(0, COMB_TAIL)],
                out_hbm.at[pl.ds(base, COMB_TAIL)],
            )

    return k(partials)


def kernel(x, edge_index):
    idx5 = edge_index.reshape(2, N_WORKERS, N_SEGS, SEG, CHUNK)
    partials = _sc_partials(x, idx5)
    return _combine(partials)


# speedup vs baseline: 1.2310x; 1.1281x over previous
import functools

import jax
import jax.numpy as jnp
from jax import lax
from jax.experimental import pallas as pl
from jax.experimental.pallas import tpu as pltpu
from jax.experimental.pallas import tpu_sc as plsc

N_NODES = 10000
D_FEAT = 128
N_EDGES = 320000

N_CORES = 2
N_TILES = 16
N_WORKERS = N_CORES * N_TILES

N_PAD = 10240
CHUNK = 80
EDGES_PER_TILE = N_EDGES // N_WORKERS
N_CHUNKS = EDGES_PER_TILE // CHUNK
SEG = 25
N_SEGS = N_CHUNKS // SEG
RING_TRIPS = (SEG - 4) // 3
ROWS_PER_TILE = N_PAD // N_TILES


def _sc_partials(x, idx5):
    mesh = plsc.VectorSubcoreMesh(core_axis_name="c", subcore_axis_name="s")

    @functools.partial(
        pl.kernel,
        mesh=mesh,
        out_type=jax.ShapeDtypeStruct((N_CORES, N_PAD, D_FEAT), jnp.float32),
        scratch_types=[
            pltpu.VMEM((SEG, CHUNK), jnp.int32),
            pltpu.VMEM((SEG, CHUNK), jnp.int32),
            pltpu.VMEM((CHUNK, D_FEAT), jnp.float32),
            pltpu.VMEM((CHUNK, D_FEAT), jnp.float32),
            pltpu.VMEM((CHUNK, D_FEAT), jnp.float32),
            pltpu.VMEM_SHARED((N_PAD, D_FEAT), jnp.float32),
            pltpu.SemaphoreType.DMA,
            pltpu.SemaphoreType.DMA,
            pltpu.SemaphoreType.DMA,
            pltpu.SemaphoreType.DMA,
            pltpu.SemaphoreType.DMA,
            pltpu.SemaphoreType.DMA,
        ],
    )
    def k(
        x_hbm, idx_hbm, out_hbm,
        src_t, dst_t, r0, r1, r2, acc,
        gs0, gs1, gs2, ss0, ss1, ss2,
    ):
        c = lax.axis_index("c")
        s = lax.axis_index("s")
        wid = s * N_CORES + c
        rows = (r0, r1, r2)
        gs = (gs0, gs1, gs2)
        ss = (ss0, ss1, ss2)

        def fire_gather(j, b):
            pltpu.async_copy(x_hbm.at[src_t.at[j]], rows[b], gs[b])

        def wait_gather(j, b):
            pltpu.make_async_copy(x_hbm.at[src_t.at[j]], rows[b], gs[b]).wait()

        def fire_scatter(j, b):
            pltpu.async_copy(rows[b], acc.at[dst_t.at[j]], ss[b], add=True)

        def wait_scatter(j, b):
            pltpu.make_async_copy(rows[b], acc.at[dst_t.at[j]], ss[b]).wait()

        pltpu.sync_copy(idx_hbm.at[0, wid, 0], src_t)
        pltpu.sync_copy(idx_hbm.at[1, wid, 0], dst_t)
        fire_gather(0, 0)
        fire_gather(1, 1)

        zv = jnp.zeros((16,), jnp.float32)

        def zbody(i, carry):
            for kk in range(8):
                r2[i, pl.ds(kk * 16, 16)] = zv
            return carry

        lax.fori_loop(0, CHUNK, zbody, 0)
        for kk in range(ROWS_PER_TILE // CHUNK):
            pltpu.sync_copy(
                r2, acc.at[pl.ds(s * ROWS_PER_TILE + kk * CHUNK, CHUNK)]
            )
        plsc.subcore_barrier()

        for seg in range(N_SEGS):
            if seg > 0:
                pltpu.sync_copy(idx_hbm.at[0, wid, seg], src_t)
                pltpu.sync_copy(idx_hbm.at[1, wid, seg], dst_t)

                fire_gather(0, 0)
                fire_gather(1, 1)

            def body(t, carry):
                for kk in range(3):
                    j = 3 * t + kk
                    wait_gather(j, kk)
                    fire_scatter(j, kk)
                    prev = (kk + 2) % 3
                    if kk == 0:
                        @pl.when(t > 0)
                        def _():
                            wait_scatter(j - 1, prev)
                    else:
                        wait_scatter(j - 1, prev)
                    fire_gather(j + 2, prev)
                return carry

            lax.fori_loop(0, RING_TRIPS, body, 0)

            for j in range(SEG - 4, SEG):
                b = j % 3
                prev = (b + 2) % 3
                wait_gather(j, b)
                fire_scatter(j, b)
                wait_scatter(j - 1, prev)
                if j + 2 < SEG:
                    fire_gather(j + 2, prev)
            wait_scatter(SEG - 1, (SEG - 1) % 3)

        plsc.subcore_barrier()
        pltpu.sync_copy(
            acc.at[pl.ds(s * ROWS_PER_TILE, ROWS_PER_TILE)],
            out_hbm.at[c, pl.ds(s * ROWS_PER_TILE, ROWS_PER_TILE)],
        )

    return k(x, idx5)


COMB_ROWS = N_PAD // N_WORKERS
COMB_PIECE = 80


def _combine(partials):
    mesh = plsc.VectorSubcoreMesh(core_axis_name="c", subcore_axis_name="s")

    @functools.partial(
        pl.kernel,
        mesh=mesh,
        out_type=jax.ShapeDtypeStruct((N_NODES, D_FEAT), jnp.float32),
        scratch_types=[
            pltpu.VMEM((COMB_ROWS, D_FEAT), jnp.float32),
            pltpu.VMEM((COMB_ROWS, D_FEAT), jnp.float32),
            pltpu.SemaphoreType.DMA,
            pltpu.SemaphoreType.DMA,
            pltpu.SemaphoreType.DMA,
        ],
    )
    def k(p_hbm, out_hbm, a0, a1, sem0, sem1, sem2):
        c = lax.axis_index("c")
        s = lax.axis_index("s")
        wid = s * N_CORES + c
        base = wid * COMB_ROWS
        np = COMB_ROWS // COMB_PIECE
        last_w = wid == N_WORKERS - 1

        def piece(p):
            sl = pl.ds(p * COMB_PIECE, COMB_PIECE)
            hsl = pl.ds(base + p * COMB_PIECE, COMB_PIECE)
            return sl, hsl

        for p in range(np):
            sl, hsl = piece(p)
            pltpu.async_copy(p_hbm.at[0, hsl], a0.at[sl], sem0)
            pltpu.async_copy(p_hbm.at[1, hsl], a1.at[sl], sem1)

        for p in range(np):
            sl, hsl = piece(p)
            pltpu.make_async_copy(p_hbm.at[0, hsl], a0.at[sl], sem0).wait()
            pltpu.make_async_copy(p_hbm.at[1, hsl], a1.at[sl], sem1).wait()

            def body(i, carry):
                for kk in range(8):
                    vsl = pl.ds(kk * 16, 16)
                    row = p * COMB_PIECE + i
                    a0[row, vsl] = a0[row, vsl] + a1[row, vsl]
                return carry

            lax.fori_loop(0, COMB_PIECE, body, 0)

            if p == 0:
                pltpu.async_copy(a0.at[sl], out_hbm.at[hsl], sem2)
            else:
                @pl.when(jnp.logical_not(last_w))
                def _():
                    pltpu.async_copy(a0.at[sl], out_hbm.at[hsl], sem2)

        for p in range(np):
            sl, hsl = piece(p)
            if p == 0:
                pltpu.make_async_copy(a0.at[sl], out_hbm.at[hsl], sem2).wait()
            else:
                @pl.when(jnp.logical_not(last_w))
                def _():
                    pltpu.make_async_copy(
                        a0.at[sl], out_hbm.at[hsl], sem2
                    ).wait()

    return k(partials)


def kernel(x, edge_index):
    idx5 = edge_index.reshape(2, N_WORKERS, N_SEGS, SEG, CHUNK)
    partials = _sc_partials(x, idx5)
    return _combine(partials)
